# baseline (speedup 1.0000x reference)
import jax
import jax.numpy as jnp
from jax.experimental import pallas as pl

_BM = 512


def _router_kernel(x_ref, mask_ref, w_ref, probs_ref, logits_ref):
    x = x_ref[...]
    w = w_ref[...]
    logits = jax.lax.dot_general(
        x.astype(jnp.bfloat16),
        w.astype(jnp.bfloat16),
        (((1,), (0,)), ((), ())),
        preferred_element_type=jnp.float32,
    )
    m = jnp.max(logits, axis=-1, keepdims=True)
    e = jnp.exp(logits - m)
    p = e / jnp.sum(e, axis=-1, keepdims=True)
    probs_ref[...] = p * mask_ref[...]
    logits_ref[...] = logits


def kernel(inputs, padding_mask, num_experts, w):
    del num_experts
    inputs = inputs.astype(jnp.float32)
    tokens, d_model = inputs.shape
    n_experts = w.shape[1]
    bm = _BM if tokens % _BM == 0 else tokens
    probs, logits = pl.pallas_call(
        _router_kernel,
        grid=(tokens // bm,),
        in_specs=[
            pl.BlockSpec((bm, d_model), lambda i: (i, 0)),
            pl.BlockSpec((bm, 1), lambda i: (i, 0)),
            pl.BlockSpec((d_model, n_experts), lambda i: (0, 0)),
        ],
        out_specs=[
            pl.BlockSpec((bm, n_experts), lambda i: (i, 0)),
            pl.BlockSpec((bm, n_experts), lambda i: (i, 0)),
        ],
        out_shape=[
            jax.ShapeDtypeStruct((tokens, n_experts), jnp.float32),
            jax.ShapeDtypeStruct((tokens, n_experts), jnp.float32),
        ],
    )(inputs, padding_mask.astype(jnp.float32), w.astype(jnp.float32))
    return (probs, logits)


# speedup vs baseline: 1.0092x; 1.0092x over previous
import jax
import jax.numpy as jnp
from jax.experimental import pallas as pl
from jax.experimental.pallas import tpu as pltpu

_BM = 1024


def _router_kernel(x_ref, mask_ref, w_ref, probs_ref, logits_ref):
    x = x_ref[...]
    w = w_ref[...]
    logits = jax.lax.dot_general(
        x.astype(jnp.bfloat16),
        w.astype(jnp.bfloat16),
        (((1,), (0,)), ((), ())),
        preferred_element_type=jnp.float32,
    )
    m = jnp.max(logits, axis=-1, keepdims=True)
    e = jnp.exp(logits - m)
    p = e / jnp.sum(e, axis=-1, keepdims=True)
    probs_ref[...] = p * mask_ref[...]
    logits_ref[...] = logits


def kernel(inputs, padding_mask, num_experts, w):
    del num_experts
    inputs = inputs.astype(jnp.float32)
    tokens, d_model = inputs.shape
    n_experts = w.shape[1]
    bm = _BM if tokens % _BM == 0 else tokens
    probs, logits = pl.pallas_call(
        _router_kernel,
        grid=(tokens // bm,),
        in_specs=[
            pl.BlockSpec((bm, d_model), lambda i: (i, 0)),
            pl.BlockSpec((bm, 1), lambda i: (i, 0)),
            pl.BlockSpec((d_model, n_experts), lambda i: (0, 0)),
        ],
        out_specs=[
            pl.BlockSpec((bm, n_experts), lambda i: (i, 0)),
            pl.BlockSpec((bm, n_experts), lambda i: (i, 0)),
        ],
        out_shape=[
            jax.ShapeDtypeStruct((tokens, n_experts), jnp.float32),
            jax.ShapeDtypeStruct((tokens, n_experts), jnp.float32),
        ],
        compiler_params=pltpu.CompilerParams(
            dimension_semantics=("parallel",),
        ),
    )(inputs, padding_mask.astype(jnp.float32), w.astype(jnp.float32))
    return (probs, logits)
